# baseline (speedup 1.0000x reference)
import functools

import jax
import jax.numpy as jnp
from jax import lax
from jax.experimental import pallas as pl
from jax.experimental.pallas import tpu as pltpu
from jax.experimental.pallas import tpu_sc as plsc

N = 10000
E = 160000
NUM_COMMON = 100000
FEAT = 256
EMB = 128
H = 256
G = 64
F2 = 128

NW = 32
N_PAD = 10240
RPW = N_PAD // NW
RPS = N_PAD // 16
CH = 128
NCH = 80
NCH_S = 40
E_PAD = 16 * NCH * CH
NB = 8

_mesh = plsc.VectorSubcoreMesh(core_axis_name="c", subcore_axis_name="s")


def _agg_scratch():
    return [
        pltpu.VMEM_SHARED((N_PAD, F2), jnp.float32),
        pltpu.VMEM((2 * NB, CH), jnp.int32),
        pltpu.VMEM((2 * NB, CH), jnp.int32),
        pltpu.VMEM((CH, F2), jnp.float32),
        pltpu.VMEM((CH, F2), jnp.float32),
        pltpu.SemaphoreType.DMA,
        pltpu.SemaphoreType.DMA,
        pltpu.SemaphoreType.DMA,
        pltpu.SemaphoreType.DMA,
        pltpu.SemaphoreType.DMA,
        pltpu.SemaphoreType.DMA,
    ]


def _edge_loop(h_ref, src_h, dst_h, idxs, idxd, rows_a, rows_b,
               sem_a, sem_b, sem_sa, sem_sb, sem_is, sem_id, accum, nch):
    nblk = nch // NB
    pltpu.sync_copy(src_h.at[pl.ds(0, NB)], idxs.at[pl.ds(0, NB)])
    pltpu.sync_copy(dst_h.at[pl.ds(0, NB)], idxd.at[pl.ds(0, NB)])
    pltpu.async_copy(h_ref.at[idxs.at[0]], rows_a, sem_a)

    def body(blk, carry):
        p = (blk % 2) * NB
        q = NB - p

        @pl.when(blk + 1 < nblk)
        def _():
            pltpu.async_copy(src_h.at[pl.ds((blk + 1) * NB, NB)],
                             idxs.at[pl.ds(q, NB)], sem_is)
            pltpu.async_copy(dst_h.at[pl.ds((blk + 1) * NB, NB)],
                             idxd.at[pl.ds(q, NB)], sem_id)

        for k in range(NB):
            rbuf, rsem, rssem = ((rows_a, sem_a, sem_sa) if k % 2 == 0
                                 else (rows_b, sem_b, sem_sb))
            nbuf, nsem, nssem = ((rows_b, sem_b, sem_sb) if k % 2 == 0
                                 else (rows_a, sem_a, sem_sa))

            def wait_nbuf_scatter():
                pltpu.make_async_copy(
                    nbuf, accum.at[idxd.at[0]], nssem).wait()

            if k == 0:
                @pl.when(blk > 0)
                def _():
                    wait_nbuf_scatter()
                pltpu.async_copy(h_ref.at[idxs.at[p + 1]], nbuf, nsem)
            elif k < NB - 1:
                wait_nbuf_scatter()
                pltpu.async_copy(h_ref.at[idxs.at[p + k + 1]], nbuf, nsem)
            else:
                @pl.when(blk + 1 < nblk)
                def _():
                    wait_nbuf_scatter()
                    pltpu.make_async_copy(
                        src_h.at[pl.ds((blk + 1) * NB, NB)],
                        idxs.at[pl.ds(q, NB)], sem_is).wait()
                    pltpu.make_async_copy(
                        dst_h.at[pl.ds((blk + 1) * NB, NB)],
                        idxd.at[pl.ds(q, NB)], sem_id).wait()
                    pltpu.async_copy(h_ref.at[idxs.at[q]], nbuf, nsem)
            pltpu.make_async_copy(h_ref.at[idxs.at[p + k]], rbuf, rsem).wait()
            pltpu.async_copy(rbuf, accum.at[idxd.at[p + k]], rssem, add=True)
        return carry

    lax.fori_loop(0, nblk, body, 0)
    pltpu.make_async_copy(rows_a, accum.at[idxd.at[0]], sem_sa).wait()
    pltpu.make_async_copy(rows_b, accum.at[idxd.at[0]], sem_sb).wait()


def _make_dual(with_emb):
    outs = (jax.ShapeDtypeStruct((N_PAD, F2), jnp.float32),
            jax.ShapeDtypeStruct((N_PAD, F2), jnp.float32))
    if with_emb:
        outs = outs + (jax.ShapeDtypeStruct((N_PAD, EMB), jnp.float32),)

    def body(h_a, h_b, src3, dst3, *rest):
        if with_emb:
            (cidx, emb_tbl, z_a, z_b, h2, accum, idxs, idxd,
             rows_a, rows_b, sem_a, sem_b, sem_sa, sem_sb,
             sem_is, sem_id, idx_v) = rest
        else:
            (z_a, z_b, accum, idxs, idxd,
             rows_a, rows_b, sem_a, sem_b, sem_sa, sem_sb,
             sem_is, sem_id, idx_v) = rest
        c = lax.axis_index("c")
        s = lax.axis_index("s")

        if with_emb:
            base = (s * 2 + c) * RPW
            pltpu.sync_copy(cidx.at[pl.ds(base, RPW)], idx_v)
            for k in range(RPW // 64):
                pltpu.async_copy(emb_tbl.at[idx_v.at[pl.ds(k * 64, 64)]],
                                 rows_a.at[pl.ds(0, 64)], sem_a).wait()
                pltpu.sync_copy(rows_a.at[pl.ds(0, 64)],
                                h2.at[pl.ds(base + k * 64, 64)])

        def run(h_ref, z_ref):
            pltpu.sync_copy(h_ref.at[pl.ds(s * RPS, RPS)],
                            accum.at[pl.ds(s * RPS, RPS)])
            plsc.subcore_barrier()
            _edge_loop(h_ref, src3.at[s], dst3.at[s], idxs, idxd,
                       rows_a, rows_b, sem_a, sem_b, sem_sa, sem_sb,
                       sem_is, sem_id, accum, NCH)
            plsc.subcore_barrier()
            pltpu.sync_copy(accum.at[pl.ds(s * RPS, RPS)],
                            z_ref.at[pl.ds(s * RPS, RPS)])

        @pl.when(c == 0)
        def _():
            run(h_a, z_a)

        @pl.when(c == 1)
        def _():
            run(h_b, z_b)

    return pl.kernel(
        body,
        out_type=outs,
        mesh=_mesh,
        scratch_types=_agg_scratch() + [pltpu.VMEM((RPW,), jnp.int32)],
    )


_agg_dual_emb = _make_dual(True)
_agg_dual = _make_dual(False)


@functools.partial(
    pl.kernel,
    out_type=(jax.ShapeDtypeStruct((N_PAD, F2), jnp.float32),
              jax.ShapeDtypeStruct((N_PAD, F2), jnp.float32)),
    mesh=_mesh,
    scratch_types=_agg_scratch(),
    compiler_params=pltpu.CompilerParams(skip_device_barrier=True),
)
def _agg_split(h, src4, dst4, z_p0, z_p1,
               accum, idxs, idxd, rows_a, rows_b,
               sem_a, sem_b, sem_sa, sem_sb, sem_is, sem_id):
    c = lax.axis_index("c")
    s = lax.axis_index("s")
    wid = c * 16 + s
    pltpu.sync_copy(h.at[pl.ds(s * RPS, RPS)],
                    accum.at[pl.ds(s * RPS, RPS)])
    plsc.subcore_barrier()
    _edge_loop(h, src4.at[wid], dst4.at[wid], idxs, idxd,
               rows_a, rows_b, sem_a, sem_b, sem_sa, sem_sb,
               sem_is, sem_id, accum, NCH_S)
    plsc.subcore_barrier()

    @pl.when(c == 0)
    def _():
        pltpu.sync_copy(accum.at[pl.ds(s * RPS, RPS)],
                        z_p0.at[pl.ds(s * RPS, RPS)])

    @pl.when(c == 1)
    def _():
        pltpu.sync_copy(accum.at[pl.ds(s * RPS, RPS)],
                        z_p1.at[pl.ds(s * RPS, RPS)])


BN = 1024
_GRID = N_PAD // BN


def _mlp0_pre_body(z0, z1, b1, w1a, w1b, tp):
    f32 = jnp.float32
    t = jnp.dot(z0[...], w1a[...], preferred_element_type=f32)
    t = t + jnp.dot(z1[...], w1b[...], preferred_element_type=f32)
    tp[...] = t + b1[...]


def _mlp0_pre(z0, z1, b1, w1a, w1b):
    nblk = pl.BlockSpec((BN, F2), lambda i: (i, 0))
    wblk = pl.BlockSpec((F2, H), lambda i: (0, 0))
    bblk = pl.BlockSpec((1, H), lambda i: (0, 0))
    return pl.pallas_call(
        _mlp0_pre_body,
        grid=(_GRID,),
        in_specs=[nblk, nblk, bblk, wblk, wblk],
        out_specs=pl.BlockSpec((BN, H), lambda i: (i, 0)),
        out_shape=jax.ShapeDtypeStruct((N_PAD, H), jnp.float32),
        compiler_params=pltpu.CompilerParams(skip_device_barrier=True),
    )(z0, z1, b1, w1a, w1b)


def _mlp0_post_body(tp, z2a, z2b, h2, w1c, w2, b2, o0, o1):
    f32 = jnp.float32
    z2 = z2a[...] + z2b[...] - h2[...]
    t = tp[...] + jnp.dot(z2, w1c[...], preferred_element_type=f32)
    t = jnp.maximum(t, 0.0)
    y = jnp.dot(t, w2[...], preferred_element_type=f32) + b2[...]
    y = jnp.maximum(y, 0.0)
    o0[...] = y[:, :H // 2]
    o1[...] = y[:, H // 2:]


def _mlp0_post(tp, z2a, z2b, h2, w1c, w2, b2):
    nblk = pl.BlockSpec((BN, F2), lambda i: (i, 0))
    wblk = pl.BlockSpec((F2, H), lambda i: (0, 0))
    bblk = pl.BlockSpec((1, H), lambda i: (0, 0))
    return pl.pallas_call(
        _mlp0_post_body,
        grid=(_GRID,),
        in_specs=[pl.BlockSpec((BN, H), lambda i: (i, 0)),
                  nblk, nblk, nblk, wblk,
                  pl.BlockSpec((H, H), lambda i: (0, 0)), bblk],
        out_specs=[
            pl.BlockSpec((BN, H // 2), lambda i: (i, 0)),
            pl.BlockSpec((BN, H // 2), lambda i: (i, 0)),
        ],
        out_shape=[jax.ShapeDtypeStruct((N_PAD, H // 2), jnp.float32)] * 2,
    )(tp, z2a, z2b, h2, w1c, w2, b2)


def _mlp1_pool_body(z0, z1, w1a, w1b, b1, w2, b2, bt, out, sum_s, cnt_s):
    i = pl.program_id(0)
    f32 = jnp.float32

    @pl.when(i == 0)
    def _():
        sum_s[...] = jnp.zeros_like(sum_s)
        cnt_s[...] = jnp.zeros_like(cnt_s)

    t = jnp.dot(z0[...], w1a[...], preferred_element_type=f32)
    t = t + jnp.dot(z1[...], w1b[...], preferred_element_type=f32)
    t = jnp.maximum(t + b1[...], 0.0)
    y = jnp.dot(t, w2[...], preferred_element_type=f32) + b2[...]
    y = jnp.maximum(y, 0.0)
    b = bt[0]
    oh = (lax.broadcasted_iota(jnp.int32, (G, BN), 0) == b).astype(f32)
    sum_s[...] += jnp.dot(oh, y, preferred_element_type=f32)
    cnt_s[...] += jnp.sum(oh, axis=1, keepdims=True)

    @pl.when(i == _GRID - 1)
    def _():
        out[...] = sum_s[...] / jnp.maximum(cnt_s[:, :1], 1.0)


def _mlp1_pool(z0, z1, w1a, w1b, b1, w2, b2, bt):
    nblk = pl.BlockSpec((BN, H // 2), lambda i: (i, 0))
    wblk = pl.BlockSpec((H // 2, H), lambda i: (0, 0))
    bblk = pl.BlockSpec((1, H), lambda i: (0, 0))
    return pl.pallas_call(
        _mlp1_pool_body,
        grid=(_GRID,),
        in_specs=[nblk, nblk, wblk, wblk, bblk,
                  pl.BlockSpec((H, H), lambda i: (0, 0)), bblk,
                  pl.BlockSpec((1, 1, BN), lambda i: (i, 0, 0))],
        out_specs=pl.BlockSpec((G, H), lambda i: (0, 0)),
        out_shape=jax.ShapeDtypeStruct((G, H), jnp.float32),
        scratch_shapes=[
            pltpu.VMEM((G, H), jnp.float32),
            pltpu.VMEM((G, 128), jnp.float32),
        ],
    )(z0, z1, w1a, w1b, b1, w2, b2, bt)


def kernel(x, common_index, edge_index, batch, emb,
           W1_0, b1_0, W2_0, b2_0, W1_1, b1_1, W2_1, b2_1):
    pad_n = N_PAD - N
    cidx = common_index.astype(jnp.int32)
    cidx_p = jnp.concatenate(
        [cidx, jnp.arange(pad_n, dtype=jnp.int32) % NUM_COMMON])

    x_p = jnp.pad(x, ((0, pad_n), (0, 0)))
    h0 = x_p[:, :F2]
    h1 = x_p[:, F2:]

    src = edge_index[0].astype(jnp.int32)
    dst = edge_index[1].astype(jnp.int32)
    pe = E_PAD - E
    ar = jnp.arange(pe, dtype=jnp.int32)
    src_p = jnp.concatenate([src, ar % N])
    dst_p = jnp.concatenate([dst, N + (ar % pad_n)])
    src3 = src_p.reshape(16, NCH, CH)
    dst3 = dst_p.reshape(16, NCH, CH)
    src4 = src_p.reshape(32, NCH_S, CH)
    dst4 = dst_p.reshape(32, NCH_S, CH)

    z0, z1, h2 = _agg_dual_emb(h0, h1, src3, dst3, cidx_p, emb)
    z2a, z2b = _agg_split(h2, src4, dst4)

    tp = _mlp0_pre(z0, z1, b1_0.reshape(1, H), W1_0[:F2], W1_0[F2:2 * F2])
    h0a, h0b = _mlp0_post(tp, z2a, z2b, h2,
                          W1_0[2 * F2:], W2_0, b2_0.reshape(1, H))

    za, zb = _agg_dual(h0a, h0b, src3, dst3)

    bt = jnp.concatenate(
        [batch.astype(jnp.int32),
         jnp.full((pad_n,), -1, jnp.int32)]).reshape(_GRID, 1, BN)
    out = _mlp1_pool(za, zb, W1_1[:H // 2], W1_1[H // 2:],
                     b1_1.reshape(1, H), W2_1, b2_1.reshape(1, H), bt)
    return out


# speedup vs baseline: 1.0227x; 1.0227x over previous
import functools

import jax
import jax.numpy as jnp
from jax import lax
from jax.experimental import pallas as pl
from jax.experimental.pallas import tpu as pltpu
from jax.experimental.pallas import tpu_sc as plsc

N = 10000
E = 160000
NUM_COMMON = 100000
FEAT = 256
EMB = 128
H = 256
G = 64
F2 = 128

NW = 32
N_PAD = 10240
RPW = N_PAD // NW
RPS = N_PAD // 16
CH = 128
NCH = 80
NCH_S = 40
E_PAD = 16 * NCH * CH
NB = 8

_mesh = plsc.VectorSubcoreMesh(core_axis_name="c", subcore_axis_name="s")


def _agg_scratch():
    return [
        pltpu.VMEM_SHARED((N_PAD, F2), jnp.float32),
        pltpu.VMEM((2 * NB, CH), jnp.int32),
        pltpu.VMEM((2 * NB, CH), jnp.int32),
        pltpu.VMEM((CH, F2), jnp.float32),
        pltpu.VMEM((CH, F2), jnp.float32),
        pltpu.SemaphoreType.DMA,
        pltpu.SemaphoreType.DMA,
        pltpu.SemaphoreType.DMA,
        pltpu.SemaphoreType.DMA,
        pltpu.SemaphoreType.DMA,
        pltpu.SemaphoreType.DMA,
    ]


def _edge_loop(h_ref, src_h, dst_h, idxs, idxd, rows_a, rows_b,
               sem_a, sem_b, sem_sa, sem_sb, sem_is, sem_id, accum, nch):
    nblk = nch // NB
    pltpu.sync_copy(src_h.at[pl.ds(0, NB)], idxs.at[pl.ds(0, NB)])
    pltpu.sync_copy(dst_h.at[pl.ds(0, NB)], idxd.at[pl.ds(0, NB)])
    pltpu.async_copy(h_ref.at[idxs.at[0]], rows_a, sem_a)

    def body(blk, carry):
        p = (blk % 2) * NB
        q = NB - p

        @pl.when(blk + 1 < nblk)
        def _():
            pltpu.async_copy(src_h.at[pl.ds((blk + 1) * NB, NB)],
                             idxs.at[pl.ds(q, NB)], sem_is)
            pltpu.async_copy(dst_h.at[pl.ds((blk + 1) * NB, NB)],
                             idxd.at[pl.ds(q, NB)], sem_id)

        for k in range(NB):
            rbuf, rsem, rssem = ((rows_a, sem_a, sem_sa) if k % 2 == 0
                                 else (rows_b, sem_b, sem_sb))
            nbuf, nsem, nssem = ((rows_b, sem_b, sem_sb) if k % 2 == 0
                                 else (rows_a, sem_a, sem_sa))

            def wait_nbuf_scatter():
                pltpu.make_async_copy(
                    nbuf, accum.at[idxd.at[0]], nssem).wait()

            if k == 0:
                @pl.when(blk > 0)
                def _():
                    wait_nbuf_scatter()
                pltpu.async_copy(h_ref.at[idxs.at[p + 1]], nbuf, nsem)
            elif k < NB - 1:
                wait_nbuf_scatter()
                pltpu.async_copy(h_ref.at[idxs.at[p + k + 1]], nbuf, nsem)
            else:
                @pl.when(blk + 1 < nblk)
                def _():
                    wait_nbuf_scatter()
                    pltpu.make_async_copy(
                        src_h.at[pl.ds((blk + 1) * NB, NB)],
                        idxs.at[pl.ds(q, NB)], sem_is).wait()
                    pltpu.make_async_copy(
                        dst_h.at[pl.ds((blk + 1) * NB, NB)],
                        idxd.at[pl.ds(q, NB)], sem_id).wait()
                    pltpu.async_copy(h_ref.at[idxs.at[q]], nbuf, nsem)
            pltpu.make_async_copy(h_ref.at[idxs.at[p + k]], rbuf, rsem).wait()
            pltpu.async_copy(rbuf, accum.at[idxd.at[p + k]], rssem, add=True)
        return carry

    lax.fori_loop(0, nblk, body, 0)
    pltpu.make_async_copy(rows_a, accum.at[idxd.at[0]], sem_sa).wait()
    pltpu.make_async_copy(rows_b, accum.at[idxd.at[0]], sem_sb).wait()


@functools.partial(
    pl.kernel,
    out_type=jax.ShapeDtypeStruct((N_PAD, EMB), jnp.float32),
    mesh=_mesh,
    scratch_types=[
        pltpu.VMEM((RPW,), jnp.int32),
        pltpu.VMEM((RPW, EMB), jnp.float32),
        pltpu.SemaphoreType.DMA,
    ],
)
def _emb_gather(cidx_hbm, emb_hbm, out_hbm, idx_v, rows_v, sem):
    c = lax.axis_index("c")
    s = lax.axis_index("s")
    base = (s * 2 + c) * RPW
    pltpu.sync_copy(cidx_hbm.at[pl.ds(base, RPW)], idx_v)
    cps = []
    for k in range(RPW // 64):
        cps.append(pltpu.async_copy(
            emb_hbm.at[idx_v.at[pl.ds(k * 64, 64)]],
            rows_v.at[pl.ds(k * 64, 64)], sem))
    for cp in cps:
        cp.wait()
    pltpu.sync_copy(rows_v, out_hbm.at[pl.ds(base, RPW)])


@functools.partial(
    pl.kernel,
    out_type=(jax.ShapeDtypeStruct((N_PAD, F2), jnp.float32),
              jax.ShapeDtypeStruct((N_PAD, F2), jnp.float32)),
    mesh=_mesh,
    scratch_types=_agg_scratch(),
)
def _agg_dual(h_a, h_b, src3, dst3, z_a, z_b,
              accum, idxs, idxd, rows_a, rows_b,
              sem_a, sem_b, sem_sa, sem_sb, sem_is, sem_id):
    c = lax.axis_index("c")
    s = lax.axis_index("s")

    def run(h_ref, z_ref):
        pltpu.sync_copy(h_ref.at[pl.ds(s * RPS, RPS)],
                        accum.at[pl.ds(s * RPS, RPS)])
        plsc.subcore_barrier()
        _edge_loop(h_ref, src3.at[s], dst3.at[s], idxs, idxd,
                   rows_a, rows_b, sem_a, sem_b, sem_sa, sem_sb,
                   sem_is, sem_id, accum, NCH)
        plsc.subcore_barrier()
        pltpu.sync_copy(accum.at[pl.ds(s * RPS, RPS)],
                        z_ref.at[pl.ds(s * RPS, RPS)])

    @pl.when(c == 0)
    def _():
        run(h_a, z_a)

    @pl.when(c == 1)
    def _():
        run(h_b, z_b)


@functools.partial(
    pl.kernel,
    out_type=(jax.ShapeDtypeStruct((N_PAD, F2), jnp.float32),
              jax.ShapeDtypeStruct((N_PAD, F2), jnp.float32)),
    mesh=_mesh,
    scratch_types=_agg_scratch(),
)
def _agg_split(h, src4, dst4, z_p0, z_p1,
               accum, idxs, idxd, rows_a, rows_b,
               sem_a, sem_b, sem_sa, sem_sb, sem_is, sem_id):
    c = lax.axis_index("c")
    s = lax.axis_index("s")
    wid = c * 16 + s
    pltpu.sync_copy(h.at[pl.ds(s * RPS, RPS)],
                    accum.at[pl.ds(s * RPS, RPS)])
    plsc.subcore_barrier()
    _edge_loop(h, src4.at[wid], dst4.at[wid], idxs, idxd,
               rows_a, rows_b, sem_a, sem_b, sem_sa, sem_sb,
               sem_is, sem_id, accum, NCH_S)
    plsc.subcore_barrier()

    @pl.when(c == 0)
    def _():
        pltpu.sync_copy(accum.at[pl.ds(s * RPS, RPS)],
                        z_p0.at[pl.ds(s * RPS, RPS)])

    @pl.when(c == 1)
    def _():
        pltpu.sync_copy(accum.at[pl.ds(s * RPS, RPS)],
                        z_p1.at[pl.ds(s * RPS, RPS)])


BN = 1024
_GRID = N_PAD // BN


def _mlp0_pre_body(z0, z1, b1, w1a, w1b, tp):
    f32 = jnp.float32
    t = jnp.dot(z0[...], w1a[...], preferred_element_type=f32)
    t = t + jnp.dot(z1[...], w1b[...], preferred_element_type=f32)
    tp[...] = t + b1[...]


def _mlp0_pre(z0, z1, b1, w1a, w1b):
    nblk = pl.BlockSpec((BN, F2), lambda i: (i, 0))
    wblk = pl.BlockSpec((F2, H), lambda i: (0, 0))
    bblk = pl.BlockSpec((1, H), lambda i: (0, 0))
    return pl.pallas_call(
        _mlp0_pre_body,
        grid=(_GRID,),
        in_specs=[nblk, nblk, bblk, wblk, wblk],
        out_specs=pl.BlockSpec((BN, H), lambda i: (i, 0)),
        out_shape=jax.ShapeDtypeStruct((N_PAD, H), jnp.float32),
        compiler_params=pltpu.CompilerParams(skip_device_barrier=True),
    )(z0, z1, b1, w1a, w1b)


def _mlp0_post_body(tp, z2a, z2b, h2, w1c, w2, b2, o0, o1):
    f32 = jnp.float32
    z2 = z2a[...] + z2b[...] - h2[...]
    t = tp[...] + jnp.dot(z2, w1c[...], preferred_element_type=f32)
    t = jnp.maximum(t, 0.0)
    y = jnp.dot(t, w2[...], preferred_element_type=f32) + b2[...]
    y = jnp.maximum(y, 0.0)
    o0[...] = y[:, :H // 2]
    o1[...] = y[:, H // 2:]


def _mlp0_post(tp, z2a, z2b, h2, w1c, w2, b2):
    nblk = pl.BlockSpec((BN, F2), lambda i: (i, 0))
    wblk = pl.BlockSpec((F2, H), lambda i: (0, 0))
    bblk = pl.BlockSpec((1, H), lambda i: (0, 0))
    return pl.pallas_call(
        _mlp0_post_body,
        grid=(_GRID,),
        in_specs=[pl.BlockSpec((BN, H), lambda i: (i, 0)),
                  nblk, nblk, nblk, wblk,
                  pl.BlockSpec((H, H), lambda i: (0, 0)), bblk],
        out_specs=[
            pl.BlockSpec((BN, H // 2), lambda i: (i, 0)),
            pl.BlockSpec((BN, H // 2), lambda i: (i, 0)),
        ],
        out_shape=[jax.ShapeDtypeStruct((N_PAD, H // 2), jnp.float32)] * 2,
    )(tp, z2a, z2b, h2, w1c, w2, b2)


def _mlp1_pool_body(z0, z1, w1a, w1b, b1, w2, b2, bt, out, sum_s, cnt_s):
    i = pl.program_id(0)
    f32 = jnp.float32

    @pl.when(i == 0)
    def _():
        sum_s[...] = jnp.zeros_like(sum_s)
        cnt_s[...] = jnp.zeros_like(cnt_s)

    t = jnp.dot(z0[...], w1a[...], preferred_element_type=f32)
    t = t + jnp.dot(z1[...], w1b[...], preferred_element_type=f32)
    t = jnp.maximum(t + b1[...], 0.0)
    y = jnp.dot(t, w2[...], preferred_element_type=f32) + b2[...]
    y = jnp.maximum(y, 0.0)
    b = bt[0]
    oh = (lax.broadcasted_iota(jnp.int32, (G, BN), 0) == b).astype(f32)
    sum_s[...] += jnp.dot(oh, y, preferred_element_type=f32)
    cnt_s[...] += jnp.sum(oh, axis=1, keepdims=True)

    @pl.when(i == _GRID - 1)
    def _():
        out[...] = sum_s[...] / jnp.maximum(cnt_s[:, :1], 1.0)


def _mlp1_pool(z0, z1, w1a, w1b, b1, w2, b2, bt):
    nblk = pl.BlockSpec((BN, H // 2), lambda i: (i, 0))
    wblk = pl.BlockSpec((H // 2, H), lambda i: (0, 0))
    bblk = pl.BlockSpec((1, H), lambda i: (0, 0))
    return pl.pallas_call(
        _mlp1_pool_body,
        grid=(_GRID,),
        in_specs=[nblk, nblk, wblk, wblk, bblk,
                  pl.BlockSpec((H, H), lambda i: (0, 0)), bblk,
                  pl.BlockSpec((1, 1, BN), lambda i: (i, 0, 0))],
        out_specs=pl.BlockSpec((G, H), lambda i: (0, 0)),
        out_shape=jax.ShapeDtypeStruct((G, H), jnp.float32),
        scratch_shapes=[
            pltpu.VMEM((G, H), jnp.float32),
            pltpu.VMEM((G, 128), jnp.float32),
        ],
    )(z0, z1, w1a, w1b, b1, w2, b2, bt)


def kernel(x, common_index, edge_index, batch, emb,
           W1_0, b1_0, W2_0, b2_0, W1_1, b1_1, W2_1, b2_1):
    pad_n = N_PAD - N
    cidx = common_index.astype(jnp.int32)
    cidx_p = jnp.concatenate(
        [cidx, jnp.arange(pad_n, dtype=jnp.int32) % NUM_COMMON])
    h2 = _emb_gather(cidx_p, emb)

    x_p = jnp.pad(x, ((0, pad_n), (0, 0)))
    h0 = x_p[:, :F2]
    h1 = x_p[:, F2:]

    src = edge_index[0].astype(jnp.int32)
    dst = edge_index[1].astype(jnp.int32)
    pe = E_PAD - E
    ar = jnp.arange(pe, dtype=jnp.int32)
    src_p = jnp.concatenate([src, ar % N])
    dst_p = jnp.concatenate([dst, N + (ar % pad_n)])
    src3 = src_p.reshape(16, NCH, CH)
    dst3 = dst_p.reshape(16, NCH, CH)
    src4 = src_p.reshape(32, NCH_S, CH)
    dst4 = dst_p.reshape(32, NCH_S, CH)

    z0, z1 = _agg_dual(h0, h1, src3, dst3)
    z2a, z2b = _agg_split(h2, src4, dst4)

    tp = _mlp0_pre(z0, z1, b1_0.reshape(1, H), W1_0[:F2], W1_0[F2:2 * F2])
    h0a, h0b = _mlp0_post(tp, z2a, z2b, h2,
                          W1_0[2 * F2:], W2_0, b2_0.reshape(1, H))

    za, zb = _agg_dual(h0a, h0b, src3, dst3)

    bt = jnp.concatenate(
        [batch.astype(jnp.int32),
         jnp.full((pad_n,), -1, jnp.int32)]).reshape(_GRID, 1, BN)
    out = _mlp1_pool(za, zb, W1_1[:H // 2], W1_1[H // 2:],
                     b1_1.reshape(1, H), W2_1, b2_1.reshape(1, H), bt)
    return out


# speedup vs baseline: 1.0264x; 1.0036x over previous
import functools

import jax
import jax.numpy as jnp
from jax import lax
from jax.experimental import pallas as pl
from jax.experimental.pallas import tpu as pltpu
from jax.experimental.pallas import tpu_sc as plsc

N = 10000
E = 160000
NUM_COMMON = 100000
FEAT = 256
EMB = 128
H = 256
G = 64
F2 = 128

NW = 32
N_PAD = 10240
RPW = N_PAD // NW
RPS = N_PAD // 16
CH = 128
NCH = 80
NCH_S = 40
E_PAD = 16 * NCH * CH
NB = 8

_mesh = plsc.VectorSubcoreMesh(core_axis_name="c", subcore_axis_name="s")


def _agg_scratch():
    return [
        pltpu.VMEM_SHARED((N_PAD, F2), jnp.float32),
        pltpu.VMEM((2 * NB, CH), jnp.int32),
        pltpu.VMEM((2 * NB, CH), jnp.int32),
        pltpu.VMEM((CH, F2), jnp.float32),
        pltpu.VMEM((CH, F2), jnp.float32),
        pltpu.SemaphoreType.DMA,
        pltpu.SemaphoreType.DMA,
        pltpu.SemaphoreType.DMA,
        pltpu.SemaphoreType.DMA,
        pltpu.SemaphoreType.DMA,
        pltpu.SemaphoreType.DMA,
    ]


def _edge_loop(h_ref, src_h, dst_h, idxs, idxd, rows_a, rows_b,
               sem_a, sem_b, sem_sa, sem_sb, sem_is, sem_id, accum, nch):
    nblk = nch // NB
    pltpu.sync_copy(src_h.at[pl.ds(0, NB)], idxs.at[pl.ds(0, NB)])
    pltpu.sync_copy(dst_h.at[pl.ds(0, NB)], idxd.at[pl.ds(0, NB)])
    pltpu.async_copy(h_ref.at[idxs.at[0]], rows_a, sem_a)

    def body(blk, carry):
        p = (blk % 2) * NB
        q = NB - p

        @pl.when(blk + 1 < nblk)
        def _():
            pltpu.async_copy(src_h.at[pl.ds((blk + 1) * NB, NB)],
                             idxs.at[pl.ds(q, NB)], sem_is)
            pltpu.async_copy(dst_h.at[pl.ds((blk + 1) * NB, NB)],
                             idxd.at[pl.ds(q, NB)], sem_id)

        for k in range(NB):
            rbuf, rsem, rssem = ((rows_a, sem_a, sem_sa) if k % 2 == 0
                                 else (rows_b, sem_b, sem_sb))
            nbuf, nsem, nssem = ((rows_b, sem_b, sem_sb) if k % 2 == 0
                                 else (rows_a, sem_a, sem_sa))

            def wait_nbuf_scatter():
                pltpu.make_async_copy(
                    nbuf, accum.at[idxd.at[0]], nssem).wait()

            if k == 0:
                @pl.when(blk > 0)
                def _():
                    wait_nbuf_scatter()
                pltpu.async_copy(h_ref.at[idxs.at[p + 1]], nbuf, nsem)
            elif k < NB - 1:
                wait_nbuf_scatter()
                pltpu.async_copy(h_ref.at[idxs.at[p + k + 1]], nbuf, nsem)
            else:
                @pl.when(blk + 1 < nblk)
                def _():
                    wait_nbuf_scatter()
                    pltpu.make_async_copy(
                        src_h.at[pl.ds((blk + 1) * NB, NB)],
                        idxs.at[pl.ds(q, NB)], sem_is).wait()
                    pltpu.make_async_copy(
                        dst_h.at[pl.ds((blk + 1) * NB, NB)],
                        idxd.at[pl.ds(q, NB)], sem_id).wait()
                    pltpu.async_copy(h_ref.at[idxs.at[q]], nbuf, nsem)
            pltpu.make_async_copy(h_ref.at[idxs.at[p + k]], rbuf, rsem).wait()
            pltpu.async_copy(rbuf, accum.at[idxd.at[p + k]], rssem, add=True)
        return carry

    lax.fori_loop(0, nblk, body, 0)
    pltpu.make_async_copy(rows_a, accum.at[idxd.at[0]], sem_sa).wait()
    pltpu.make_async_copy(rows_b, accum.at[idxd.at[0]], sem_sb).wait()


@functools.partial(
    pl.kernel,
    out_type=jax.ShapeDtypeStruct((N_PAD, EMB), jnp.float32),
    mesh=_mesh,
    scratch_types=[
        pltpu.VMEM((RPW,), jnp.int32),
        pltpu.VMEM((RPW, EMB), jnp.float32),
        pltpu.SemaphoreType.DMA,
    ],
)
def _emb_gather(cidx_hbm, emb_hbm, out_hbm, idx_v, rows_v, sem):
    c = lax.axis_index("c")
    s = lax.axis_index("s")
    base = (s * 2 + c) * RPW
    pltpu.sync_copy(cidx_hbm.at[pl.ds(base, RPW)], idx_v)
    cps = []
    for k in range(RPW // 64):
        cps.append(pltpu.async_copy(
            emb_hbm.at[idx_v.at[pl.ds(k * 64, 64)]],
            rows_v.at[pl.ds(k * 64, 64)], sem))
    for cp in cps:
        cp.wait()
    pltpu.sync_copy(rows_v, out_hbm.at[pl.ds(base, RPW)])


@functools.partial(
    pl.kernel,
    out_type=(jax.ShapeDtypeStruct((N_PAD, F2), jnp.float32),
              jax.ShapeDtypeStruct((N_PAD, F2), jnp.float32)),
    mesh=_mesh,
    scratch_types=_agg_scratch(),
)
def _agg_dual(h_a, h_b, src3, dst3, z_a, z_b,
              accum, idxs, idxd, rows_a, rows_b,
              sem_a, sem_b, sem_sa, sem_sb, sem_is, sem_id):
    c = lax.axis_index("c")
    s = lax.axis_index("s")

    def run(h_ref, z_ref):
        pltpu.sync_copy(h_ref.at[pl.ds(s * RPS, RPS)],
                        accum.at[pl.ds(s * RPS, RPS)])
        plsc.subcore_barrier()
        _edge_loop(h_ref, src3.at[s], dst3.at[s], idxs, idxd,
                   rows_a, rows_b, sem_a, sem_b, sem_sa, sem_sb,
                   sem_is, sem_id, accum, NCH)
        plsc.subcore_barrier()
        pltpu.sync_copy(accum.at[pl.ds(s * RPS, RPS)],
                        z_ref.at[pl.ds(s * RPS, RPS)])

    @pl.when(c == 0)
    def _():
        run(h_a, z_a)

    @pl.when(c == 1)
    def _():
        run(h_b, z_b)


@functools.partial(
    pl.kernel,
    out_type=(jax.ShapeDtypeStruct((N_PAD, F2), jnp.float32),
              jax.ShapeDtypeStruct((N_PAD, F2), jnp.float32)),
    mesh=_mesh,
    scratch_types=_agg_scratch(),
)
def _agg_split(h, src4, dst4, z_p0, z_p1,
               accum, idxs, idxd, rows_a, rows_b,
               sem_a, sem_b, sem_sa, sem_sb, sem_is, sem_id):
    c = lax.axis_index("c")
    s = lax.axis_index("s")
    wid = c * 16 + s
    pltpu.sync_copy(h.at[pl.ds(s * RPS, RPS)],
                    accum.at[pl.ds(s * RPS, RPS)])
    plsc.subcore_barrier()
    _edge_loop(h, src4.at[wid], dst4.at[wid], idxs, idxd,
               rows_a, rows_b, sem_a, sem_b, sem_sa, sem_sb,
               sem_is, sem_id, accum, NCH_S)
    plsc.subcore_barrier()

    @pl.when(c == 0)
    def _():
        pltpu.sync_copy(accum.at[pl.ds(s * RPS, RPS)],
                        z_p0.at[pl.ds(s * RPS, RPS)])

    @pl.when(c == 1)
    def _():
        pltpu.sync_copy(accum.at[pl.ds(s * RPS, RPS)],
                        z_p1.at[pl.ds(s * RPS, RPS)])


BN = 1024
_GRID = N_PAD // BN


def _mlp0_body(z0, z1, z2a, z2b, h2, w1a, w1b, w1c, b1, w2, b2, o0, o1):
    f32 = jnp.float32
    t = jnp.dot(z0[...], w1a[...], preferred_element_type=f32)
    t = t + jnp.dot(z1[...], w1b[...], preferred_element_type=f32)
    z2 = z2a[...] + z2b[...] - h2[...]
    t = t + jnp.dot(z2, w1c[...], preferred_element_type=f32)
    t = jnp.maximum(t + b1[...], 0.0)
    y = jnp.dot(t, w2[...], preferred_element_type=f32) + b2[...]
    y = jnp.maximum(y, 0.0)
    o0[...] = y[:, :H // 2]
    o1[...] = y[:, H // 2:]


def _mlp0(z0, z1, z2a, z2b, h2, w1a, w1b, w1c, b1, w2, b2):
    nblk = pl.BlockSpec((BN, F2), lambda i: (i, 0))
    wblk = pl.BlockSpec((F2, H), lambda i: (0, 0))
    bblk = pl.BlockSpec((1, H), lambda i: (0, 0))
    return pl.pallas_call(
        _mlp0_body,
        grid=(_GRID,),
        in_specs=[nblk, nblk, nblk, nblk, nblk, wblk, wblk, wblk, bblk,
                  pl.BlockSpec((H, H), lambda i: (0, 0)), bblk],
        out_specs=[
            pl.BlockSpec((BN, H // 2), lambda i: (i, 0)),
            pl.BlockSpec((BN, H // 2), lambda i: (i, 0)),
        ],
        out_shape=[jax.ShapeDtypeStruct((N_PAD, H // 2), jnp.float32)] * 2,
    )(z0, z1, z2a, z2b, h2, w1a, w1b, w1c, b1, w2, b2)


def _mlp1_pool_body(z0, z1, w1a, w1b, b1, w2, b2, bt, out, sum_s, cnt_s):
    i = pl.program_id(0)
    f32 = jnp.float32

    @pl.when(i == 0)
    def _():
        sum_s[...] = jnp.zeros_like(sum_s)
        cnt_s[...] = jnp.zeros_like(cnt_s)

    t = jnp.dot(z0[...], w1a[...], preferred_element_type=f32)
    t = t + jnp.dot(z1[...], w1b[...], preferred_element_type=f32)
    t = jnp.maximum(t + b1[...], 0.0)
    y = jnp.dot(t, w2[...], preferred_element_type=f32) + b2[...]
    y = jnp.maximum(y, 0.0)
    b = bt[0]
    oh = (lax.broadcasted_iota(jnp.int32, (G, BN), 0) == b).astype(f32)
    sum_s[...] += jnp.dot(oh, y, preferred_element_type=f32)
    cnt_s[...] += jnp.sum(oh, axis=1, keepdims=True)

    @pl.when(i == _GRID - 1)
    def _():
        out[...] = sum_s[...] / jnp.maximum(cnt_s[:, :1], 1.0)


def _mlp1_pool(z0, z1, w1a, w1b, b1, w2, b2, bt):
    nblk = pl.BlockSpec((BN, H // 2), lambda i: (i, 0))
    wblk = pl.BlockSpec((H // 2, H), lambda i: (0, 0))
    bblk = pl.BlockSpec((1, H), lambda i: (0, 0))
    return pl.pallas_call(
        _mlp1_pool_body,
        grid=(_GRID,),
        in_specs=[nblk, nblk, wblk, wblk, bblk,
                  pl.BlockSpec((H, H), lambda i: (0, 0)), bblk,
                  pl.BlockSpec((1, 1, BN), lambda i: (i, 0, 0))],
        out_specs=pl.BlockSpec((G, H), lambda i: (0, 0)),
        out_shape=jax.ShapeDtypeStruct((G, H), jnp.float32),
        scratch_shapes=[
            pltpu.VMEM((G, H), jnp.float32),
            pltpu.VMEM((G, 128), jnp.float32),
        ],
    )(z0, z1, w1a, w1b, b1, w2, b2, bt)


def kernel(x, common_index, edge_index, batch, emb,
           W1_0, b1_0, W2_0, b2_0, W1_1, b1_1, W2_1, b2_1):
    pad_n = N_PAD - N
    cidx = common_index.astype(jnp.int32)
    cidx_p = jnp.concatenate(
        [cidx, jnp.arange(pad_n, dtype=jnp.int32) % NUM_COMMON])
    h2 = _emb_gather(cidx_p, emb)

    x_p = jnp.pad(x, ((0, pad_n), (0, 0)))
    h0 = x_p[:, :F2]
    h1 = x_p[:, F2:]

    src = edge_index[0].astype(jnp.int32)
    dst = edge_index[1].astype(jnp.int32)
    pe = E_PAD - E
    ar = jnp.arange(pe, dtype=jnp.int32)
    src_p = jnp.concatenate([src, ar % N])
    dst_p = jnp.concatenate([dst, N + (ar % pad_n)])
    src3 = src_p.reshape(16, NCH, CH)
    dst3 = dst_p.reshape(16, NCH, CH)
    src4 = src_p.reshape(32, NCH_S, CH)
    dst4 = dst_p.reshape(32, NCH_S, CH)

    z0, z1 = _agg_dual(h0, h1, src3, dst3)
    z2a, z2b = _agg_split(h2, src4, dst4)

    h0a, h0b = _mlp0(z0, z1, z2a, z2b, h2,
                     W1_0[:F2], W1_0[F2:2 * F2], W1_0[2 * F2:],
                     b1_0.reshape(1, H), W2_0, b2_0.reshape(1, H))

    za, zb = _agg_dual(h0a, h0b, src3, dst3)

    bt = jnp.concatenate(
        [batch.astype(jnp.int32),
         jnp.full((pad_n,), -1, jnp.int32)]).reshape(_GRID, 1, BN)
    out = _mlp1_pool(za, zb, W1_1[:H // 2], W1_1[H // 2:],
                     b1_1.reshape(1, H), W2_1, b2_1.reshape(1, H), bt)
    return out
